# Initial kernel scaffold; baseline (speedup 1.0000x reference)
import jax
import jax.numpy as jnp
from jax.experimental import pallas as pl


def kernel(x):
    raise NotImplementedError("write your pallas kernel here")


# speedup vs baseline: 15.2912x; 15.2912x over previous
import functools

import jax
import jax.numpy as jnp
from jax import lax
from jax.experimental import pallas as pl
from jax.experimental.pallas import tpu as pltpu
from jax.experimental.pallas import tpu_sc as plsc

ROWS, COLS = 128, 32768
N = ROWS * COLS
K = int(0.1 * N)
NB = 65536
NC, NS = 2, 16
NW = NC * NS
PER_W = N // NW
CHUNK = 8192
NCHUNK = PER_W // CHUNK
INT_MIN32 = -2147483648


def _sortable_key(f16):
    u = lax.bitcast_convert_type(f16, jnp.int32)
    m = lax.shift_right_arithmetic(u, 31)
    flip = lax.bitwise_or(m, jnp.full((16,), INT_MIN32, jnp.int32))
    return lax.bitwise_xor(u, flip)


def _sc_hist_body(x_hbm, sel_hbm, out_hbm, data_v, hist_v, sel_v, *, low_pass):
    c = lax.axis_index("c")
    s = lax.axis_index("s")
    wid = s * NC + c
    zeros = jnp.zeros((16,), jnp.int32)

    @pl.loop(0, NB // 16, unroll=8)
    def _zero(i):
        hist_v[pl.ds(i * 16, 16)] = zeros

    if low_pass:
        pltpu.sync_copy(sel_hbm.at[pl.ds(0, 16)], sel_v)
        bvec = sel_v[...]
    low_mask = jnp.full((16,), 0xFFFF, jnp.int32)
    base = wid * PER_W

    @pl.loop(0, NCHUNK)
    def _chunk(ci):
        pltpu.sync_copy(x_hbm.at[pl.ds(base + ci * CHUNK, CHUNK)], data_v)

        @pl.loop(0, CHUNK // 16, unroll=4)
        def _vec(vi):
            key = _sortable_key(data_v[pl.ds(vi * 16, 16)])
            hi = lax.shift_right_logical(key, 16)
            if low_pass:
                lo = lax.bitwise_and(key, low_mask)
                cnt, lastm = plsc.scan_count(lo, mask=hi == bvec)
                plsc.addupdate_scatter(hist_v, [lo], cnt, mask=lastm)
            else:
                cnt, lastm = plsc.scan_count(hi)
                plsc.addupdate_scatter(hist_v, [hi], cnt, mask=lastm)

    pltpu.sync_copy(hist_v, out_hbm.at[pl.ds(wid * NB, NB)])


@functools.lru_cache(maxsize=None)
def _sc_kernels():
    mesh = plsc.VectorSubcoreMesh(
        core_axis_name="c", subcore_axis_name="s", num_cores=NC, num_subcores=NS
    )

    @functools.partial(
        pl.kernel,
        out_type=jax.ShapeDtypeStruct((NW * NB,), jnp.int32),
        mesh=mesh,
        scratch_types=[
            pltpu.VMEM((CHUNK,), jnp.float32),
            pltpu.VMEM((NB,), jnp.int32),
        ],
        compiler_params=pltpu.CompilerParams(needs_layout_passes=False),
    )
    def _sc_hist_hi(x_hbm, out_hbm, data_v, hist_v):
        _sc_hist_body(x_hbm, None, out_hbm, data_v, hist_v, None, low_pass=False)

    @functools.partial(
        pl.kernel,
        out_type=jax.ShapeDtypeStruct((NW * NB,), jnp.int32),
        mesh=mesh,
        scratch_types=[
            pltpu.VMEM((CHUNK,), jnp.float32),
            pltpu.VMEM((NB,), jnp.int32),
            pltpu.VMEM((16,), jnp.int32),
        ],
        compiler_params=pltpu.CompilerParams(needs_layout_passes=False),
    )
    def _sc_hist_lo(x_hbm, sel_hbm, out_hbm, data_v, hist_v, sel_v):
        _sc_hist_body(x_hbm, sel_hbm, out_hbm, data_v, hist_v, sel_v, low_pass=True)

    return _sc_hist_hi, _sc_hist_lo


def _cum_incl(hf):
    i0 = lax.broadcasted_iota(jnp.int32, (512, 512), 0)
    i1 = lax.broadcasted_iota(jnp.int32, (512, 512), 1)
    l_strict = (i1 < i0).astype(jnp.float32)
    j0 = lax.broadcasted_iota(jnp.int32, (128, 128), 0)
    j1 = lax.broadcasted_iota(jnp.int32, (128, 128), 1)
    u_incl = (j0 <= j1).astype(jnp.float32)
    prev_rows = jnp.sum(
        jax.lax.dot(l_strict, hf, preferred_element_type=jnp.float32), axis=1
    )
    col_cum = jax.lax.dot(hf, u_incl, preferred_element_type=jnp.float32)
    return prev_rows[:, None] + col_cum


def _rank_select(h32, rank_f):
    hf = jnp.sum(h32, axis=0).astype(jnp.float32).reshape(512, 128)
    cum = _cum_incl(hf)
    bstar = jnp.sum((cum <= rank_f).astype(jnp.float32)).astype(jnp.int32)
    idx = (
        lax.broadcasted_iota(jnp.int32, (512, 128), 0) * 128
        + lax.broadcasted_iota(jnp.int32, (512, 128), 1)
    )
    below = jnp.sum(jnp.where(idx < bstar, hf, 0.0)).astype(jnp.int32)
    return bstar, below


def _tca_body(h_ref, sel_ref):
    bstar, below = _rank_select(h_ref[...], jnp.float32(K))
    r = K - below
    row = lax.broadcasted_iota(jnp.int32, (8, 128), 0)
    sel_ref[...] = jnp.where(row == 0, bstar, jnp.where(row == 1, r, 0))


_tca = pl.pallas_call(
    _tca_body,
    out_shape=jax.ShapeDtypeStruct((8, 128), jnp.int32),
)


def _tcb_body(sel_ref, h_ref, x_ref, out_ref, eps_ref):
    @pl.when(pl.program_id(0) == 0)
    def _():
        bstar = sel_ref[0, 0]
        r = sel_ref[1, 0]
        lstar, _ = _rank_select(h_ref[...], r.astype(jnp.float32))
        key = lax.bitwise_or(lax.shift_left(bstar, 16), lstar)
        u = jnp.where(
            key < 0,
            lax.bitwise_xor(key, jnp.int32(INT_MIN32)),
            lax.bitwise_not(key),
        )
        eps_ref[0] = lax.bitcast_convert_type(u, jnp.float32)

    out_ref[...] = jnp.maximum(x_ref[...], eps_ref[0])


_tcb = pl.pallas_call(
    _tcb_body,
    grid=(16,),
    in_specs=[
        pl.BlockSpec((8, 128), lambda i: (0, 0), memory_space=pltpu.SMEM),
        pl.BlockSpec((NW, NB), lambda i: (0, 0)),
        pl.BlockSpec((ROWS // 16, COLS), lambda i: (i, 0)),
    ],
    out_specs=pl.BlockSpec((ROWS // 16, COLS), lambda i: (i, 0)),
    out_shape=jax.ShapeDtypeStruct((ROWS, COLS), jnp.float32),
    scratch_shapes=[pltpu.SMEM((1,), jnp.float32)],
)


def kernel(x):
    sc_hist_hi, sc_hist_lo = _sc_kernels()
    xf = x.reshape(-1)
    h1 = sc_hist_hi(xf)
    sel = _tca(h1.reshape(NW, NB))
    h2 = sc_hist_lo(xf, sel.reshape(-1))
    return _tcb(sel, h2.reshape(NW, NB), x)


# speedup vs baseline: 48.9955x; 3.2042x over previous
import functools

import jax
import jax.numpy as jnp
from jax import lax
from jax.experimental import pallas as pl
from jax.experimental.pallas import tpu as pltpu
from jax.experimental.pallas import tpu_sc as plsc

ROWS, COLS = 128, 32768
N = ROWS * COLS
K = int(0.1 * N)
NB = 65536
NC, NS = 2, 16
NW = NC * NS
PER_W = N // NW
CHUNK = 32768
NCHUNK = PER_W // CHUNK
INT_MIN32 = -2147483648


def _sortable_key(f16):
    u = lax.bitcast_convert_type(f16, jnp.int32)
    m = lax.shift_right_arithmetic(u, 31)
    flip = lax.bitwise_or(m, jnp.full((16,), INT_MIN32, jnp.int32))
    return lax.bitwise_xor(u, flip)


def _sc_hist_body(x_hbm, sel_hbm, out_hbm, data_v, hist_v, sel_v, *, low_pass):
    c = lax.axis_index("c")
    s = lax.axis_index("s")
    wid = s * NC + c
    zeros = jnp.zeros((16,), jnp.int32)

    @pl.loop(0, NB // 16, unroll=8)
    def _zero(i):
        hist_v[pl.ds(i * 16, 16)] = zeros

    if low_pass:
        pltpu.sync_copy(sel_hbm.at[pl.ds(0, 16)], sel_v)
        bvec = sel_v[...]
    low_mask = jnp.full((16,), 0xFFFF, jnp.int32)
    base = wid * PER_W

    @pl.loop(0, NCHUNK)
    def _chunk(ci):
        pltpu.sync_copy(x_hbm.at[pl.ds(base + ci * CHUNK, CHUNK)], data_v)

        @plsc.parallel_loop(0, CHUNK // 16, unroll=8)
        def _vec(vi):
            key = _sortable_key(data_v[pl.ds(vi * 16, 16)])
            hi = lax.shift_right_logical(key, 16)
            if low_pass:
                lo = lax.bitwise_and(key, low_mask)
                cnt, lastm = plsc.scan_count(lo, mask=hi == bvec)
                plsc.addupdate_scatter(hist_v, [lo], cnt, mask=lastm)
            else:
                cnt, lastm = plsc.scan_count(hi)
                plsc.addupdate_scatter(hist_v, [hi], cnt, mask=lastm)

    pltpu.sync_copy(hist_v, out_hbm.at[pl.ds(wid * NB, NB)])


@functools.lru_cache(maxsize=None)
def _sc_kernels():
    mesh = plsc.VectorSubcoreMesh(
        core_axis_name="c", subcore_axis_name="s", num_cores=NC, num_subcores=NS
    )

    @functools.partial(
        pl.kernel,
        out_type=jax.ShapeDtypeStruct((NW * NB,), jnp.int32),
        mesh=mesh,
        scratch_types=[
            pltpu.VMEM((CHUNK,), jnp.float32),
            pltpu.VMEM((NB,), jnp.int32),
        ],
        compiler_params=pltpu.CompilerParams(needs_layout_passes=False),
    )
    def _sc_hist_hi(x_hbm, out_hbm, data_v, hist_v):
        _sc_hist_body(x_hbm, None, out_hbm, data_v, hist_v, None, low_pass=False)

    @functools.partial(
        pl.kernel,
        out_type=jax.ShapeDtypeStruct((NW * NB,), jnp.int32),
        mesh=mesh,
        scratch_types=[
            pltpu.VMEM((CHUNK,), jnp.float32),
            pltpu.VMEM((NB,), jnp.int32),
            pltpu.VMEM((16,), jnp.int32),
        ],
        compiler_params=pltpu.CompilerParams(needs_layout_passes=False),
    )
    def _sc_hist_lo(x_hbm, sel_hbm, out_hbm, data_v, hist_v, sel_v):
        _sc_hist_body(x_hbm, sel_hbm, out_hbm, data_v, hist_v, sel_v, low_pass=True)

    return _sc_hist_hi, _sc_hist_lo


def _cum_incl(hf):
    i0 = lax.broadcasted_iota(jnp.int32, (512, 512), 0)
    i1 = lax.broadcasted_iota(jnp.int32, (512, 512), 1)
    l_strict = (i1 < i0).astype(jnp.float32)
    j0 = lax.broadcasted_iota(jnp.int32, (128, 128), 0)
    j1 = lax.broadcasted_iota(jnp.int32, (128, 128), 1)
    u_incl = (j0 <= j1).astype(jnp.float32)
    prev_rows = jnp.sum(
        jax.lax.dot(l_strict, hf, preferred_element_type=jnp.float32), axis=1
    )
    col_cum = jax.lax.dot(hf, u_incl, preferred_element_type=jnp.float32)
    return prev_rows[:, None] + col_cum


def _rank_select(h32, rank_f):
    hf = jnp.sum(h32, axis=0).astype(jnp.float32).reshape(512, 128)
    cum = _cum_incl(hf)
    bstar = jnp.sum((cum <= rank_f).astype(jnp.float32)).astype(jnp.int32)
    idx = (
        lax.broadcasted_iota(jnp.int32, (512, 128), 0) * 128
        + lax.broadcasted_iota(jnp.int32, (512, 128), 1)
    )
    below = jnp.sum(jnp.where(idx < bstar, hf, 0.0)).astype(jnp.int32)
    return bstar, below


def _tca_body(h_ref, sel_ref):
    bstar, below = _rank_select(h_ref[...], jnp.float32(K))
    r = K - below
    row = lax.broadcasted_iota(jnp.int32, (8, 128), 0)
    sel_ref[...] = jnp.where(row == 0, bstar, jnp.where(row == 1, r, 0))


_tca = pl.pallas_call(
    _tca_body,
    out_shape=jax.ShapeDtypeStruct((8, 128), jnp.int32),
)


def _tcb_body(sel_ref, h_ref, x_ref, out_ref, eps_ref):
    @pl.when(pl.program_id(0) == 0)
    def _():
        bstar = sel_ref[0, 0]
        r = sel_ref[1, 0]
        lstar, _ = _rank_select(h_ref[...], r.astype(jnp.float32))
        key = lax.bitwise_or(lax.shift_left(bstar, 16), lstar)
        u = jnp.where(
            key < 0,
            lax.bitwise_xor(key, jnp.int32(INT_MIN32)),
            lax.bitwise_not(key),
        )
        eps_ref[0] = lax.bitcast_convert_type(u, jnp.float32)

    out_ref[...] = jnp.maximum(x_ref[...], eps_ref[0])


_tcb = pl.pallas_call(
    _tcb_body,
    grid=(16,),
    in_specs=[
        pl.BlockSpec((8, 128), lambda i: (0, 0), memory_space=pltpu.SMEM),
        pl.BlockSpec((NW, NB), lambda i: (0, 0)),
        pl.BlockSpec((ROWS // 16, COLS), lambda i: (i, 0)),
    ],
    out_specs=pl.BlockSpec((ROWS // 16, COLS), lambda i: (i, 0)),
    out_shape=jax.ShapeDtypeStruct((ROWS, COLS), jnp.float32),
    scratch_shapes=[pltpu.SMEM((1,), jnp.float32)],
)


def kernel(x):
    sc_hist_hi, sc_hist_lo = _sc_kernels()
    xf = x.reshape(-1)
    h1 = sc_hist_hi(xf)
    sel = _tca(h1.reshape(NW, NB))
    h2 = sc_hist_lo(xf, sel.reshape(-1))
    return _tcb(sel, h2.reshape(NW, NB), x)


# speedup vs baseline: 56.3696x; 1.1505x over previous
import functools

import jax
import jax.numpy as jnp
from jax import lax
from jax.experimental import pallas as pl
from jax.experimental.pallas import tpu as pltpu
from jax.experimental.pallas import tpu_sc as plsc

ROWS, COLS = 128, 32768
N = ROWS * COLS
K = int(0.1 * N)
NB = 65536
NC, NS = 2, 16
NW = NC * NS
PER_W = N // NW
CHUNK = 32768
NCHUNK = PER_W // CHUNK
INT_MIN32 = -2147483648


def _sortable_key(f16):
    u = lax.bitcast_convert_type(f16, jnp.int32)
    m = lax.shift_right_arithmetic(u, 31)
    flip = lax.bitwise_or(m, jnp.full((16,), INT_MIN32, jnp.int32))
    return lax.bitwise_xor(u, flip)


def _sc_hist_body(x_hbm, sel_hbm, out_hbm, data_v, hist_v, sel_v, *, low_pass):
    c = lax.axis_index("c")
    s = lax.axis_index("s")
    wid = s * NC + c
    zeros = jnp.zeros((16,), jnp.int32)

    @pl.loop(0, NB // 16, unroll=8)
    def _zero(i):
        hist_v[pl.ds(i * 16, 16)] = zeros

    if low_pass:
        pltpu.sync_copy(sel_hbm.at[pl.ds(0, 16)], sel_v)
        bvec = sel_v[...]
    low_mask = jnp.full((16,), 0xFFFF, jnp.int32)
    base = wid * PER_W

    @pl.loop(0, NCHUNK)
    def _chunk(ci):
        pltpu.sync_copy(x_hbm.at[pl.ds(base + ci * CHUNK, CHUNK)], data_v)

        @plsc.parallel_loop(0, CHUNK // 16, unroll=8)
        def _vec(vi):
            key = _sortable_key(data_v[pl.ds(vi * 16, 16)])
            hi = lax.shift_right_logical(key, 16)
            if low_pass:
                lo = lax.bitwise_and(key, low_mask)
                cnt, lastm = plsc.scan_count(lo, mask=hi == bvec)
                plsc.addupdate_scatter(hist_v, [lo], cnt, mask=lastm)
            else:
                cnt, lastm = plsc.scan_count(hi)
                plsc.addupdate_scatter(hist_v, [hi], cnt, mask=lastm)

    pltpu.sync_copy(hist_v, out_hbm.at[pl.ds(wid * NB, NB)])


@functools.lru_cache(maxsize=None)
def _sc_kernels():
    mesh = plsc.VectorSubcoreMesh(
        core_axis_name="c", subcore_axis_name="s", num_cores=NC, num_subcores=NS
    )

    @functools.partial(
        pl.kernel,
        out_type=jax.ShapeDtypeStruct((NW * NB,), jnp.int32),
        mesh=mesh,
        scratch_types=[
            pltpu.VMEM((CHUNK,), jnp.float32),
            pltpu.VMEM((NB,), jnp.int32),
        ],
        compiler_params=pltpu.CompilerParams(needs_layout_passes=False),
    )
    def _sc_hist_hi(x_hbm, out_hbm, data_v, hist_v):
        _sc_hist_body(x_hbm, None, out_hbm, data_v, hist_v, None, low_pass=False)

    @functools.partial(
        pl.kernel,
        out_type=jax.ShapeDtypeStruct((NW * NB,), jnp.int32),
        mesh=mesh,
        scratch_types=[
            pltpu.VMEM((CHUNK,), jnp.float32),
            pltpu.VMEM((NB,), jnp.int32),
            pltpu.VMEM((16,), jnp.int32),
        ],
        compiler_params=pltpu.CompilerParams(needs_layout_passes=False),
    )
    def _sc_hist_lo(x_hbm, sel_hbm, out_hbm, data_v, hist_v, sel_v):
        _sc_hist_body(x_hbm, sel_hbm, out_hbm, data_v, hist_v, sel_v, low_pass=True)

    return _sc_hist_hi, _sc_hist_lo


def _cum_incl(hf):
    i0 = lax.broadcasted_iota(jnp.int32, (512, 512), 0)
    i1 = lax.broadcasted_iota(jnp.int32, (512, 512), 1)
    l_strict = (i1 < i0).astype(jnp.float32)
    j0 = lax.broadcasted_iota(jnp.int32, (128, 128), 0)
    j1 = lax.broadcasted_iota(jnp.int32, (128, 128), 1)
    u_incl = (j0 <= j1).astype(jnp.float32)
    prev_rows = jnp.sum(
        jax.lax.dot(l_strict, hf, preferred_element_type=jnp.float32), axis=1
    )
    col_cum = jax.lax.dot(hf, u_incl, preferred_element_type=jnp.float32)
    return prev_rows[:, None] + col_cum


def _rank_select(h32, rank_f):
    hf = jnp.sum(h32.reshape(NW, 512, 128), axis=0).astype(jnp.float32)
    cum = _cum_incl(hf)
    bstar = jnp.sum((cum <= rank_f).astype(jnp.float32)).astype(jnp.int32)
    idx = (
        lax.broadcasted_iota(jnp.int32, (512, 128), 0) * 128
        + lax.broadcasted_iota(jnp.int32, (512, 128), 1)
    )
    below = jnp.sum(jnp.where(idx < bstar, hf, 0.0)).astype(jnp.int32)
    return bstar, below


def _tca_body(h_ref, sel_ref):
    bstar, below = _rank_select(h_ref[...], jnp.float32(K))
    r = K - below
    row = lax.broadcasted_iota(jnp.int32, (8, 128), 0)
    sel_ref[...] = jnp.where(row == 0, bstar, jnp.where(row == 1, r, 0))


_tca = pl.pallas_call(
    _tca_body,
    out_shape=jax.ShapeDtypeStruct((8, 128), jnp.int32),
)


def _tcb_body(sel_ref, h_ref, x_ref, out_ref, eps_ref):
    @pl.when(pl.program_id(0) == 0)
    def _():
        bstar = sel_ref[0, 0]
        r = sel_ref[1, 0]
        lstar, _ = _rank_select(h_ref[...], r.astype(jnp.float32))
        key = lax.bitwise_or(lax.shift_left(bstar, 16), lstar)
        u = jnp.where(
            key < 0,
            lax.bitwise_xor(key, jnp.int32(INT_MIN32)),
            lax.bitwise_not(key),
        )
        eps_ref[0] = lax.bitcast_convert_type(u, jnp.float32)

    out_ref[...] = jnp.maximum(x_ref[...], eps_ref[0])


_tcb = pl.pallas_call(
    _tcb_body,
    grid=(16,),
    in_specs=[
        pl.BlockSpec((8, 128), lambda i: (0, 0), memory_space=pltpu.SMEM),
        pl.BlockSpec((NW * NB // 128, 128), lambda i: (0, 0)),
        pl.BlockSpec((ROWS // 16, COLS), lambda i: (i, 0)),
    ],
    out_specs=pl.BlockSpec((ROWS // 16, COLS), lambda i: (i, 0)),
    out_shape=jax.ShapeDtypeStruct((ROWS, COLS), jnp.float32),
    scratch_shapes=[pltpu.SMEM((1,), jnp.float32)],
)


def kernel(x):
    sc_hist_hi, sc_hist_lo = _sc_kernels()
    xf = x.reshape(-1)
    h1 = sc_hist_hi(xf)
    sel = _tca(h1.reshape(NW * NB // 128, 128))
    h2 = sc_hist_lo(xf, sel.reshape(-1))
    return _tcb(sel, h2.reshape(NW * NB // 128, 128), x)


# speedup vs baseline: 63.6406x; 1.1290x over previous
import functools

import jax
import jax.numpy as jnp
from jax import lax
from jax.experimental import pallas as pl
from jax.experimental.pallas import tpu as pltpu
from jax.experimental.pallas import tpu_sc as plsc

ROWS, COLS = 128, 32768
N = ROWS * COLS
K = int(0.1 * N)
NB = 65536
NC, NS = 2, 16
NW = NC * NS
PER_W = N // NW
CCOLS = 2048
NCHUNK = (COLS // 2) // CCOLS
INT_MIN32 = -2147483648


def _sortable_key(f16):
    u = lax.bitcast_convert_type(f16, jnp.int32)
    m = lax.shift_right_arithmetic(u, 31)
    flip = lax.bitwise_or(m, jnp.full((16,), INT_MIN32, jnp.int32))
    return lax.bitwise_xor(u, flip)


def _sc_hist_body(x_hbm, sel_hbm, out_hbm, data_v, hist_v, sel_v, *, low_pass):
    c = lax.axis_index("c")
    s = lax.axis_index("s")
    wid = s * NC + c
    zeros = jnp.zeros((16,), jnp.int32)

    @pl.loop(0, NB // 16, unroll=8)
    def _zero(i):
        hist_v[pl.ds(i * 16, 16)] = zeros

    if low_pass:
        pltpu.sync_copy(sel_hbm.at[pl.ds(0, 16)], sel_v)
        bvec = sel_v[...]
    low_mask = jnp.full((16,), 0xFFFF, jnp.int32)
    band = wid // 2
    col0 = (wid % 2) * (COLS // 2)

    @pl.loop(0, NCHUNK)
    def _chunk(ci):
        pltpu.sync_copy(
            x_hbm.at[pl.ds(band * 8, 8), pl.ds(col0 + ci * CCOLS, CCOLS)],
            data_v,
        )

        @plsc.parallel_loop(0, 8 * CCOLS // 16, unroll=8)
        def _vec(vi):
            r = lax.shift_right_logical(vi, 7)
            c = lax.shift_left(lax.bitwise_and(vi, 127), 4)
            key = _sortable_key(data_v[r, pl.ds(c, 16)])
            hi = lax.shift_right_logical(key, 16)
            if low_pass:
                lo = lax.bitwise_and(key, low_mask)
                cnt, lastm = plsc.scan_count(lo, mask=hi == bvec)
                plsc.addupdate_scatter(hist_v, [lo], cnt, mask=lastm)
            else:
                cnt, lastm = plsc.scan_count(hi)
                plsc.addupdate_scatter(hist_v, [hi], cnt, mask=lastm)

    pltpu.sync_copy(hist_v, out_hbm.at[pl.ds(wid * NB, NB)])


@functools.lru_cache(maxsize=None)
def _sc_kernels():
    mesh = plsc.VectorSubcoreMesh(
        core_axis_name="c", subcore_axis_name="s", num_cores=NC, num_subcores=NS
    )

    @functools.partial(
        pl.kernel,
        out_type=jax.ShapeDtypeStruct((NW * NB,), jnp.int32),
        mesh=mesh,
        scratch_types=[
            pltpu.VMEM((8, CCOLS), jnp.float32),
            pltpu.VMEM((NB,), jnp.int32),
        ],
        compiler_params=pltpu.CompilerParams(
            needs_layout_passes=False, use_tc_tiling_on_sc=True
        ),
    )
    def _sc_hist_hi(x_hbm, out_hbm, data_v, hist_v):
        _sc_hist_body(x_hbm, None, out_hbm, data_v, hist_v, None, low_pass=False)

    @functools.partial(
        pl.kernel,
        out_type=jax.ShapeDtypeStruct((NW * NB,), jnp.int32),
        mesh=mesh,
        scratch_types=[
            pltpu.VMEM((8, CCOLS), jnp.float32),
            pltpu.VMEM((NB,), jnp.int32),
            pltpu.VMEM((16,), jnp.int32),
        ],
        compiler_params=pltpu.CompilerParams(
            needs_layout_passes=False, use_tc_tiling_on_sc=True
        ),
    )
    def _sc_hist_lo(x_hbm, sel_hbm, out_hbm, data_v, hist_v, sel_v):
        _sc_hist_body(x_hbm, sel_hbm, out_hbm, data_v, hist_v, sel_v, low_pass=True)

    return _sc_hist_hi, _sc_hist_lo


def _cum_incl(hf):
    i0 = lax.broadcasted_iota(jnp.int32, (512, 512), 0)
    i1 = lax.broadcasted_iota(jnp.int32, (512, 512), 1)
    l_strict = (i1 < i0).astype(jnp.float32)
    j0 = lax.broadcasted_iota(jnp.int32, (128, 128), 0)
    j1 = lax.broadcasted_iota(jnp.int32, (128, 128), 1)
    u_incl = (j0 <= j1).astype(jnp.float32)
    prev_rows = jnp.sum(
        jax.lax.dot(l_strict, hf, preferred_element_type=jnp.float32), axis=1
    )
    col_cum = jax.lax.dot(hf, u_incl, preferred_element_type=jnp.float32)
    return prev_rows[:, None] + col_cum


def _rank_select(h32, rank_f):
    hf = jnp.sum(h32.reshape(NW, 512, 128), axis=0).astype(jnp.float32)
    cum = _cum_incl(hf)
    bstar = jnp.sum((cum <= rank_f).astype(jnp.float32)).astype(jnp.int32)
    idx = (
        lax.broadcasted_iota(jnp.int32, (512, 128), 0) * 128
        + lax.broadcasted_iota(jnp.int32, (512, 128), 1)
    )
    below = jnp.sum(jnp.where(idx < bstar, hf, 0.0)).astype(jnp.int32)
    return bstar, below


def _tca_body(h_ref, sel_ref):
    bstar, below = _rank_select(h_ref[...], jnp.float32(K))
    r = K - below
    row = lax.broadcasted_iota(jnp.int32, (8, 128), 0)
    sel_ref[...] = jnp.where(row == 0, bstar, jnp.where(row == 1, r, 0))


_tca = pl.pallas_call(
    _tca_body,
    out_shape=jax.ShapeDtypeStruct((8, 128), jnp.int32),
)


def _tcb_body(sel_ref, h_ref, x_ref, out_ref, eps_ref):
    @pl.when(pl.program_id(0) == 0)
    def _():
        bstar = sel_ref[0, 0]
        r = sel_ref[1, 0]
        lstar, _ = _rank_select(h_ref[...], r.astype(jnp.float32))
        key = lax.bitwise_or(lax.shift_left(bstar, 16), lstar)
        u = jnp.where(
            key < 0,
            lax.bitwise_xor(key, jnp.int32(INT_MIN32)),
            lax.bitwise_not(key),
        )
        eps_ref[0] = lax.bitcast_convert_type(u, jnp.float32)

    out_ref[...] = jnp.maximum(x_ref[...], eps_ref[0])


_tcb = pl.pallas_call(
    _tcb_body,
    grid=(16,),
    in_specs=[
        pl.BlockSpec((8, 128), lambda i: (0, 0), memory_space=pltpu.SMEM),
        pl.BlockSpec((NW * NB // 128, 128), lambda i: (0, 0)),
        pl.BlockSpec((ROWS // 16, COLS), lambda i: (i, 0)),
    ],
    out_specs=pl.BlockSpec((ROWS // 16, COLS), lambda i: (i, 0)),
    out_shape=jax.ShapeDtypeStruct((ROWS, COLS), jnp.float32),
    scratch_shapes=[pltpu.SMEM((1,), jnp.float32)],
)


def kernel(x):
    sc_hist_hi, sc_hist_lo = _sc_kernels()
    h1 = sc_hist_hi(x)
    sel = _tca(h1.reshape(NW * NB // 128, 128))
    h2 = sc_hist_lo(x, sel.reshape(-1))
    return _tcb(sel, h2.reshape(NW * NB // 128, 128), x)


# speedup vs baseline: 77.3131x; 1.2148x over previous
import functools

import jax
import jax.numpy as jnp
from jax import lax
from jax.experimental import pallas as pl
from jax.experimental.pallas import tpu as pltpu
from jax.experimental.pallas import tpu_sc as plsc

ROWS, COLS = 128, 32768
N = ROWS * COLS
K = int(0.1 * N)
NB = 65536
NC, NS = 2, 16
NW = NC * NS
PER_W = N // NW
CCOLS = 2048
NCHUNK = (COLS // 2) // CCOLS
INT_MIN32 = -2147483648


def _sortable_key(f16):
    u = lax.bitcast_convert_type(f16, jnp.int32)
    m = lax.shift_right_arithmetic(u, 31)
    flip = lax.bitwise_or(m, jnp.full((16,), INT_MIN32, jnp.int32))
    return lax.bitwise_xor(u, flip)


def _sc_hist_body(
    x_hbm, sel_hbm, out_hbm, data0_v, data1_v, sem0, sem1, hist_v, sel_v, *, low_pass
):
    c = lax.axis_index("c")
    s = lax.axis_index("s")
    wid = s * NC + c
    zeros = jnp.zeros((16,), jnp.int32)
    low_mask = jnp.full((16,), 0xFFFF, jnp.int32)
    band = wid // 2
    col0 = (wid % 2) * (COLS // 2)

    def _chunk_src(ci):
        return x_hbm.at[pl.ds(band * 8, 8), pl.ds(col0 + ci * CCOLS, CCOLS)]

    def _issue(ci, buf, sem):
        pltpu.make_async_copy(_chunk_src(ci), buf, sem).start()

    def _wait(buf, sem):
        pltpu.make_async_copy(_chunk_src(0), buf, sem).wait()

    _issue(0, data0_v, sem0)
    _issue(1, data1_v, sem1)

    @pl.loop(0, NB // 16, unroll=8)
    def _zero(i):
        hist_v[pl.ds(i * 16, 16)] = zeros

    if low_pass:
        pltpu.sync_copy(sel_hbm.at[pl.ds(0, 16)], sel_v)
        bvec = sel_v[...]

    def _compute(buf):
        @plsc.parallel_loop(0, 8 * CCOLS // 16, unroll=8)
        def _vec(vi):
            r = lax.shift_right_logical(vi, 7)
            cc = lax.shift_left(lax.bitwise_and(vi, 127), 4)
            key = _sortable_key(buf[r, pl.ds(cc, 16)])
            hi = lax.shift_right_logical(key, 16)
            if low_pass:
                lo = lax.bitwise_and(key, low_mask)
                cnt, lastm = plsc.scan_count(lo, mask=hi == bvec)
                plsc.addupdate_scatter(hist_v, [lo], cnt, mask=lastm)
            else:
                cnt, lastm = plsc.scan_count(hi)
                plsc.addupdate_scatter(hist_v, [hi], cnt, mask=lastm)

    @pl.loop(0, NCHUNK // 2)
    def _pair(i):
        c0 = 2 * i
        _wait(data0_v, sem0)
        _compute(data0_v)
        _issue(jnp.minimum(c0 + 2, NCHUNK - 1), data0_v, sem0)
        _wait(data1_v, sem1)
        _compute(data1_v)
        _issue(jnp.minimum(c0 + 3, NCHUNK - 1), data1_v, sem1)

    _wait(data0_v, sem0)
    _wait(data1_v, sem1)

    pltpu.sync_copy(hist_v, out_hbm.at[pl.ds(wid * NB, NB)])


@functools.lru_cache(maxsize=None)
def _sc_kernels():
    mesh = plsc.VectorSubcoreMesh(
        core_axis_name="c", subcore_axis_name="s", num_cores=NC, num_subcores=NS
    )

    @functools.partial(
        pl.kernel,
        out_type=jax.ShapeDtypeStruct((NW * NB,), jnp.int32),
        mesh=mesh,
        scratch_types=[
            pltpu.VMEM((8, CCOLS), jnp.float32),
            pltpu.VMEM((8, CCOLS), jnp.float32),
            pltpu.SemaphoreType.DMA,
            pltpu.SemaphoreType.DMA,
            pltpu.VMEM((NB,), jnp.int32),
        ],
        compiler_params=pltpu.CompilerParams(
            needs_layout_passes=False, use_tc_tiling_on_sc=True
        ),
    )
    def _sc_hist_hi(x_hbm, out_hbm, data0_v, data1_v, sem0, sem1, hist_v):
        _sc_hist_body(
            x_hbm, None, out_hbm, data0_v, data1_v, sem0, sem1, hist_v, None,
            low_pass=False,
        )

    @functools.partial(
        pl.kernel,
        out_type=jax.ShapeDtypeStruct((NW * NB,), jnp.int32),
        mesh=mesh,
        scratch_types=[
            pltpu.VMEM((8, CCOLS), jnp.float32),
            pltpu.VMEM((8, CCOLS), jnp.float32),
            pltpu.SemaphoreType.DMA,
            pltpu.SemaphoreType.DMA,
            pltpu.VMEM((NB,), jnp.int32),
            pltpu.VMEM((16,), jnp.int32),
        ],
        compiler_params=pltpu.CompilerParams(
            needs_layout_passes=False, use_tc_tiling_on_sc=True
        ),
    )
    def _sc_hist_lo(x_hbm, sel_hbm, out_hbm, data0_v, data1_v, sem0, sem1, hist_v, sel_v):
        _sc_hist_body(
            x_hbm, sel_hbm, out_hbm, data0_v, data1_v, sem0, sem1, hist_v, sel_v,
            low_pass=True,
        )

    return _sc_hist_hi, _sc_hist_lo


def _cum_incl(hf):
    i0 = lax.broadcasted_iota(jnp.int32, (512, 512), 0)
    i1 = lax.broadcasted_iota(jnp.int32, (512, 512), 1)
    l_strict = (i1 < i0).astype(jnp.float32)
    j0 = lax.broadcasted_iota(jnp.int32, (128, 128), 0)
    j1 = lax.broadcasted_iota(jnp.int32, (128, 128), 1)
    u_incl = (j0 <= j1).astype(jnp.float32)
    prev_rows = jnp.sum(
        jax.lax.dot(l_strict, hf, preferred_element_type=jnp.float32), axis=1
    )
    col_cum = jax.lax.dot(hf, u_incl, preferred_element_type=jnp.float32)
    return prev_rows[:, None] + col_cum


def _rank_select(h32, rank_f):
    hf = jnp.sum(h32.reshape(NW, 512, 128), axis=0).astype(jnp.float32)
    cum = _cum_incl(hf)
    bstar = jnp.sum((cum <= rank_f).astype(jnp.float32)).astype(jnp.int32)
    idx = (
        lax.broadcasted_iota(jnp.int32, (512, 128), 0) * 128
        + lax.broadcasted_iota(jnp.int32, (512, 128), 1)
    )
    below = jnp.sum(jnp.where(idx < bstar, hf, 0.0)).astype(jnp.int32)
    return bstar, below


def _tca_body(h_ref, sel_ref):
    bstar, below = _rank_select(h_ref[...], jnp.float32(K))
    r = K - below
    row = lax.broadcasted_iota(jnp.int32, (8, 128), 0)
    sel_ref[...] = jnp.where(row == 0, bstar, jnp.where(row == 1, r, 0))


_tca = pl.pallas_call(
    _tca_body,
    out_shape=jax.ShapeDtypeStruct((8, 128), jnp.int32),
)


def _tcb_body(sel_ref, h_ref, x_ref, out_ref, eps_ref):
    @pl.when(pl.program_id(0) == 0)
    def _():
        bstar = sel_ref[0, 0]
        r = sel_ref[1, 0]
        lstar, _ = _rank_select(h_ref[...], r.astype(jnp.float32))
        key = lax.bitwise_or(lax.shift_left(bstar, 16), lstar)
        u = jnp.where(
            key < 0,
            lax.bitwise_xor(key, jnp.int32(INT_MIN32)),
            lax.bitwise_not(key),
        )
        eps_ref[0] = lax.bitcast_convert_type(u, jnp.float32)

    out_ref[...] = jnp.maximum(x_ref[...], eps_ref[0])


_tcb = pl.pallas_call(
    _tcb_body,
    grid=(16,),
    in_specs=[
        pl.BlockSpec((8, 128), lambda i: (0, 0), memory_space=pltpu.SMEM),
        pl.BlockSpec((NW * NB // 128, 128), lambda i: (0, 0)),
        pl.BlockSpec((ROWS // 16, COLS), lambda i: (i, 0)),
    ],
    out_specs=pl.BlockSpec((ROWS // 16, COLS), lambda i: (i, 0)),
    out_shape=jax.ShapeDtypeStruct((ROWS, COLS), jnp.float32),
    scratch_shapes=[pltpu.SMEM((1,), jnp.float32)],
)


def kernel(x):
    sc_hist_hi, sc_hist_lo = _sc_kernels()
    h1 = sc_hist_hi(x)
    sel = _tca(h1.reshape(NW * NB // 128, 128))
    h2 = sc_hist_lo(x, sel.reshape(-1))
    return _tcb(sel, h2.reshape(NW * NB // 128, 128), x)
